# baseline (speedup 1.0000x reference)
import functools

import jax
import jax.numpy as jnp
from jax import lax
from jax.experimental import pallas as pl
from jax.experimental.pallas import tpu as pltpu
from jax.experimental.pallas import tpu_sc as plsc

N = 10000
E = 160000
D = 256
DE = 16
DEP = 128

NC = 2
NS = 16
NW = NC * NS
NPW = 320
NP2 = NW * NPW
CAP = 6400
ES = NW * CAP
GCH = 160
ECH = 512
BCH = 8000
NBCH = E // BCH
SCH = 128

_mesh = plsc.VectorSubcoreMesh(
    core_axis_name="c", subcore_axis_name="s", num_cores=NC, num_subcores=NS)


def _wid():
    return lax.axis_index("s") * NC + lax.axis_index("c")


@functools.partial(
    pl.kernel,
    out_type=(
        jax.ShapeDtypeStruct((ES,), jnp.int32),
        jax.ShapeDtypeStruct((ES,), jnp.int32),
        jax.ShapeDtypeStruct((ES,), jnp.int32),
        jax.ShapeDtypeStruct((NW * 16,), jnp.int32),
    ),
    mesh=_mesh,
    compiler_params=pltpu.CompilerParams(needs_layout_passes=False),
    scratch_types=[
        pltpu.VMEM((BCH,), jnp.int32),
        pltpu.VMEM((BCH,), jnp.int32),
        pltpu.VMEM((CAP,), jnp.int32),
        pltpu.VMEM((CAP,), jnp.int32),
        pltpu.VMEM((CAP,), jnp.int32),
        pltpu.VMEM((CAP,), jnp.int32),
        pltpu.VMEM((16,), jnp.int32),
    ],
)
def _bucketize(dst_hbm, src_hbm, lists_hbm, offs_hbm, srcp_hbm, counts_hbm,
               dbuf, sbuf, mpack, spack, mlist, molist, cbuf):
    w = _wid()
    lo = w * NPW
    hi = lo + NPW
    iota = lax.iota(jnp.int32, 16)
    trash = jnp.full((16,), NPW, jnp.int32)
    zeros = jnp.zeros((16,), jnp.int32)

    def init_body(i, _):
        mpack[pl.ds(i * 16, 16)] = trash
        spack[pl.ds(i * 16, 16)] = zeros
        return 0
    lax.fori_loop(0, CAP // 16, init_body, 0)

    def chunk_body(c, cnt):
        pltpu.sync_copy(dst_hbm.at[pl.ds(c * BCH, BCH)], dbuf)
        pltpu.sync_copy(src_hbm.at[pl.ds(c * BCH, BCH)], sbuf)

        def vec_body(v, cnt):
            d = dbuf[pl.ds(v * 16, 16)]
            s = sbuf[pl.ds(v * 16, 16)]
            eid = c * BCH + v * 16 + iota
            m = (d >= lo) & (d < hi)
            packed = jnp.where(m, (eid << 9) | (d - lo), trash)
            sval = jnp.where(m, s, zeros)
            key = jnp.where(m, iota, iota + 16)
            _, sv = plsc.sort_key_val(key, packed)
            _, ss = plsc.sort_key_val(key, sval)
            mpack[pl.ds(cnt, 16)] = sv
            spack[pl.ds(cnt, 16)] = ss
            pc = plsc.all_reduce_population_count(m)
            return cnt + pc[0]
        return lax.fori_loop(0, BCH // 16, vec_body, cnt)

    cnt = lax.fori_loop(0, NBCH, chunk_body, jnp.int32(0))
    mpack[pl.ds(cnt, 16)] = trash
    spack[pl.ds(cnt, 16)] = zeros

    def unpack_body(i, _):
        v = mpack[pl.ds(i * 16, 16)]
        mlist[pl.ds(i * 16, 16)] = v >> 9
        molist[pl.ds(i * 16, 16)] = v & 511
        return 0
    lax.fori_loop(0, CAP // 16, unpack_body, 0)

    cbuf[pl.ds(0, 16)] = jnp.full((16,), cnt, jnp.int32)
    base = pl.multiple_of(w * CAP, 8)
    pltpu.sync_copy(mlist, lists_hbm.at[pl.ds(base, CAP)])
    pltpu.sync_copy(molist, offs_hbm.at[pl.ds(base, CAP)])
    pltpu.sync_copy(spack, srcp_hbm.at[pl.ds(base, CAP)])
    pltpu.sync_copy(cbuf, counts_hbm.at[pl.ds(pl.multiple_of(w * 16, 8), 16)])


@functools.partial(
    pl.kernel,
    out_type=jax.ShapeDtypeStruct((ES, DEP), jnp.float32),
    mesh=_mesh,
    scratch_types=[
        pltpu.VMEM((ECH,), jnp.int32),
        pltpu.VMEM((ECH, DEP), jnp.float32),
        pltpu.VMEM((16,), jnp.int32),
        pltpu.SemaphoreType.DMA,
    ],
)
def _permute_ea(ea_hbm, lists_hbm, counts_hbm, eap_hbm, idx, eabuf, cbuf, sem):
    w = _wid()
    base = pl.multiple_of(w * CAP, 8)
    pltpu.sync_copy(counts_hbm.at[pl.ds(pl.multiple_of(w * 16, 8), 16)], cbuf)
    cnt = cbuf[pl.ds(0, 16)][0]
    nch = (cnt + ECH - 1) // ECH

    def body(k, _):
        cb = pl.multiple_of(k * ECH, 8)
        pltpu.sync_copy(lists_hbm.at[pl.ds(base + cb, ECH)], idx)
        pltpu.async_copy(ea_hbm.at[idx], eabuf, sem).wait()
        pltpu.sync_copy(eabuf, eap_hbm.at[pl.ds(base + cb, ECH)])
        return 0
    lax.fori_loop(0, nch, body, 0)


@functools.partial(
    pl.kernel,
    out_type=jax.ShapeDtypeStruct((ES, D), jnp.float32),
    mesh=_mesh,
    scratch_types=[
        pltpu.VMEM((NPW + 1, D), jnp.float32),
        pltpu.VMEM((GCH,), jnp.int32),
        pltpu.VMEM((GCH,), jnp.int32),
        pltpu.VMEM((GCH, D), jnp.float32),
        pltpu.VMEM((16,), jnp.int32),
        pltpu.SemaphoreType.DMA,
        pltpu.SemaphoreType.DMA,
    ],
)
def _gather_add(pd_hbm, ps_hbm, srcp_hbm, offs_hbm, counts_hbm, pre_hbm,
                pdloc, idx, offv, rows, cbuf, semg, semw):
    w = _wid()
    nlo = pl.multiple_of(w * NPW, 8)
    base = pl.multiple_of(w * CAP, 8)
    pltpu.sync_copy(pd_hbm.at[pl.ds(nlo, NPW)], pdloc.at[pl.ds(0, NPW)])
    pltpu.sync_copy(counts_hbm.at[pl.ds(pl.multiple_of(w * 16, 8), 16)], cbuf)
    cnt = cbuf[pl.ds(0, 16)][0]
    nch = (cnt + GCH - 1) // GCH

    def body(k, _):
        cb = pl.multiple_of(k * GCH, 8)
        pltpu.sync_copy(srcp_hbm.at[pl.ds(base + cb, GCH)], idx)
        pltpu.sync_copy(offs_hbm.at[pl.ds(base + cb, GCH)], offv)

        @pl.when(k > 0)
        def _():
            pltpu.make_async_copy(
                rows, pre_hbm.at[pl.ds(base, GCH)], semw).wait()

        pltpu.async_copy(ps_hbm.at[idx], rows, semg).wait()

        def add_body(g, _):
            off_vec = offv[pl.ds(g * 16, 16)]
            for l in range(16):
                off = off_vec[l]
                i = g * 16 + l
                for j in range(D // 16):
                    rows[i, pl.ds(j * 16, 16)] = (
                        rows[i, pl.ds(j * 16, 16)]
                        + pdloc[off, pl.ds(j * 16, 16)])
            return 0
        lax.fori_loop(0, GCH // 16, add_body, 0)
        pltpu.async_copy(rows, pre_hbm.at[pl.ds(base + cb, GCH)], semw)
        return 0
    lax.fori_loop(0, nch, body, 0)

    @pl.when(cnt > 0)
    def _():
        pltpu.make_async_copy(rows, pre_hbm.at[pl.ds(base, GCH)], semw).wait()


@functools.partial(
    pl.kernel,
    out_type=jax.ShapeDtypeStruct((NP2, D), jnp.float32),
    mesh=_mesh,
    scratch_types=[
        pltpu.VMEM((NPW + 1, D), jnp.float32),
        pltpu.VMEM((SCH,), jnp.int32),
        pltpu.VMEM((SCH, D), jnp.float32),
        pltpu.VMEM((16,), jnp.int32),
    ],
)
def _scatter_max(m2_hbm, offs_hbm, counts_hbm, agg_hbm,
                 table, offb, rows, cbuf):
    w = _wid()
    base = pl.multiple_of(w * CAP, 8)

    def zero_body(r, _):
        for j in range(D // 16):
            table[r, pl.ds(j * 16, 16)] = jnp.zeros((16,), jnp.float32)
        return 0
    lax.fori_loop(0, NPW + 1, zero_body, 0)

    pltpu.sync_copy(counts_hbm.at[pl.ds(pl.multiple_of(w * 16, 8), 16)], cbuf)
    cnt = cbuf[pl.ds(0, 16)][0]
    nchunks = (cnt + SCH - 1) // SCH

    def chunk_body(k, _):
        cb = pl.multiple_of(k * SCH, 8)
        pltpu.sync_copy(offs_hbm.at[pl.ds(base + cb, SCH)], offb)
        pltpu.sync_copy(m2_hbm.at[pl.ds(base + cb, SCH)], rows)

        def group_body(g, _):
            off_vec = offb[pl.ds(g * 16, 16)]
            for l in range(16):
                off = off_vec[l]
                i = g * 16 + l
                for j in range(D // 16):
                    cur = table[off, pl.ds(j * 16, 16)]
                    seg = rows[i, pl.ds(j * 16, 16)]
                    table[off, pl.ds(j * 16, 16)] = jnp.maximum(cur, seg)
            return 0
        lax.fori_loop(0, SCH // 16, group_body, 0)
        return 0
    lax.fori_loop(0, nchunks, chunk_body, 0)

    lo = pl.multiple_of(w * NPW, 8)
    pltpu.sync_copy(table.at[pl.ds(0, NPW)], agg_hbm.at[pl.ds(lo, NPW)])


def _node_transform_body(h_ref, wa1_ref, wa2_ref, pd_ref, ps_ref):
    h = h_ref[...]
    a = jnp.dot(h, wa1_ref[...], preferred_element_type=jnp.float32)
    b = jnp.dot(h, wa2_ref[...], preferred_element_type=jnp.float32)
    pd_ref[...] = a - b
    ps_ref[...] = b


def _node_transform(h, wa1, wa2):
    bn = 1024
    grid = NP2 // bn
    return pl.pallas_call(
        _node_transform_body,
        grid=(grid,),
        in_specs=[
            pl.BlockSpec((bn, D), lambda i: (i, 0)),
            pl.BlockSpec((D, D), lambda i: (0, 0)),
            pl.BlockSpec((D, D), lambda i: (0, 0)),
        ],
        out_specs=[
            pl.BlockSpec((bn, D), lambda i: (i, 0)),
            pl.BlockSpec((bn, D), lambda i: (i, 0)),
        ],
        out_shape=[
            jax.ShapeDtypeStruct((NP2, D), jnp.float32),
            jax.ShapeDtypeStruct((NP2, D), jnp.float32),
        ],
    )(h, wa1, wa2)


def _edge_mlp_body(pre_ref, ea_ref, wa3_ref, ba_ref, wb_ref, bb_ref, out_ref):
    ea = jnp.dot(ea_ref[...], wa3_ref[...], preferred_element_type=jnp.float32)
    a = jax.nn.relu(pre_ref[...] + ea + ba_ref[...])
    out = jnp.dot(a, wb_ref[...], preferred_element_type=jnp.float32)
    out_ref[...] = jax.nn.relu(out + bb_ref[...])


def _edge_mlp(pre, eap, wa3, ba, wb, bb):
    be = 512
    grid = ES // be
    ba2 = ba.reshape(1, -1)
    bb2 = bb.reshape(1, -1)
    return pl.pallas_call(
        _edge_mlp_body,
        grid=(grid,),
        in_specs=[
            pl.BlockSpec((be, D), lambda i: (i, 0)),
            pl.BlockSpec((be, DEP), lambda i: (i, 0)),
            pl.BlockSpec((DEP, D), lambda i: (0, 0)),
            pl.BlockSpec((1, D), lambda i: (0, 0)),
            pl.BlockSpec((D, D), lambda i: (0, 0)),
            pl.BlockSpec((1, D), lambda i: (0, 0)),
        ],
        out_specs=pl.BlockSpec((be, D), lambda i: (i, 0)),
        out_shape=jax.ShapeDtypeStruct((ES, D), jnp.float32),
    )(pre, eap, wa3, ba2, wb, bb2)


def _head_body(h_ref, w3_ref, b3_ref, w4_ref, b4_ref, out_ref):
    a = jax.nn.relu(
        jnp.dot(h_ref[...], w3_ref[...], preferred_element_type=jnp.float32)
        + b3_ref[...])
    o = jnp.dot(a, w4_ref[...], preferred_element_type=jnp.float32) + b4_ref[...]
    out_ref[...] = jax.nn.sigmoid(o)


def _head(h, w3, b3, w4, b4):
    bn = 1000
    grid = N // bn
    h2 = w3.shape[1]
    out = w4.shape[1]
    return pl.pallas_call(
        _head_body,
        grid=(grid,),
        in_specs=[
            pl.BlockSpec((bn, D), lambda i: (i, 0)),
            pl.BlockSpec((D, h2), lambda i: (0, 0)),
            pl.BlockSpec((1, h2), lambda i: (0, 0)),
            pl.BlockSpec((h2, out), lambda i: (0, 0)),
            pl.BlockSpec((1, out), lambda i: (0, 0)),
        ],
        out_specs=pl.BlockSpec((bn, out), lambda i: (i, 0)),
        out_shape=jax.ShapeDtypeStruct((N, out), jnp.float32),
    )(h, w3, b3.reshape(1, -1), w4, b4.reshape(1, -1))


def _layer(h, srcp, offs, counts, eap, wa, ba, wb, bb, hin):
    wa1 = wa[:hin]
    wa2 = wa[hin:2 * hin]
    wa3 = jnp.concatenate(
        [wa[2 * hin:], jnp.zeros((DEP - DE, wa.shape[1]), wa.dtype)], axis=0)
    pd, ps = _node_transform(h, wa1, wa2)
    pre = _gather_add(pd, ps, srcp, offs, counts)
    m2 = _edge_mlp(pre, eap, wa3, ba, wb, bb)
    return _scatter_max(m2, offs, counts)


def kernel(x, edge_index, edge_attr, W1a, b1a, W1b, b1b, W2a, b2a, W2b, b2b,
           W3, b3, W4, b4):
    src = edge_index[0]
    dst = edge_index[1]
    lists, offs, srcp, counts = _bucketize(dst, src)
    ea128 = jnp.concatenate(
        [edge_attr, jnp.zeros((E, DEP - DE), edge_attr.dtype)], axis=1)
    eap = _permute_ea(ea128, lists, counts)
    xp = jnp.concatenate(
        [x, jnp.zeros((NP2 - N, D), jnp.float32)], axis=0)
    h = _layer(xp, srcp, offs, counts, eap, W1a, b1a, W1b, b1b, D)
    h = _layer(h, srcp, offs, counts, eap, W2a, b2a, W2b, b2b, D)
    return _head(h[:N], W3, b3, W4, b4)


# speedup vs baseline: 1.6439x; 1.6439x over previous
import functools

import jax
import jax.numpy as jnp
from jax import lax
from jax.experimental import pallas as pl
from jax.experimental.pallas import tpu as pltpu
from jax.experimental.pallas import tpu_sc as plsc

N = 10000
E = 160000
D = 256
DE = 16

NC = 2
NS = 16
NW = NC * NS
NPW = 320
EPW = E // NW
GCH = 200
NGCH = EPW // GCH
BCH = 8000
NBCH = E // BCH
CAP = 6144
SCH = 128

_mesh = plsc.VectorSubcoreMesh(
    core_axis_name="c", subcore_axis_name="s", num_cores=NC, num_subcores=NS)


def _wid():
    return lax.axis_index("s") * NC + lax.axis_index("c")


@functools.partial(
    pl.kernel,
    out_type=(
        jax.ShapeDtypeStruct((NW, CAP), jnp.int32),
        jax.ShapeDtypeStruct((NW, CAP), jnp.int32),
        jax.ShapeDtypeStruct((NW, 16), jnp.int32),
    ),
    mesh=_mesh,
    compiler_params=pltpu.CompilerParams(needs_layout_passes=False),
    scratch_types=[
        pltpu.VMEM((BCH,), jnp.int32),
        pltpu.VMEM((CAP,), jnp.int32),
        pltpu.VMEM((CAP,), jnp.int32),
        pltpu.VMEM((CAP,), jnp.int32),
        pltpu.VMEM((16,), jnp.int32),
    ],
)
def _bucketize(dst_hbm, lists_hbm, offs_hbm, counts_hbm,
               dbuf, mpack, mlist, molist, cbuf):
    w = _wid()
    lo = w * NPW
    hi = lo + NPW
    iota = lax.iota(jnp.int32, 16)
    trash = jnp.full((16,), NPW, jnp.int32)

    def init_body(i, _):
        mpack[pl.ds(i * 16, 16)] = trash
        return 0
    lax.fori_loop(0, CAP // 16, init_body, 0)

    def chunk_body(c, cnt):
        pltpu.sync_copy(dst_hbm.at[pl.ds(c * BCH, BCH)], dbuf)

        def vec_body(v, cnt):
            d = dbuf[pl.ds(v * 16, 16)]
            eid = c * BCH + v * 16 + iota
            m = (d >= lo) & (d < hi)
            packed = jnp.where(m, (eid << 9) | (d - lo), trash)
            key = jnp.where(m, iota, iota + 16)
            _, sv = plsc.sort_key_val(key, packed)
            mpack[pl.ds(cnt, 16)] = sv
            pc = plsc.all_reduce_population_count(m)
            return cnt + pc[0]
        return lax.fori_loop(0, BCH // 16, vec_body, cnt)

    cnt = lax.fori_loop(0, NBCH, chunk_body, jnp.int32(0))
    mpack[pl.ds(cnt, 16)] = trash

    def unpack_body(i, _):
        v = mpack[pl.ds(i * 16, 16)]
        mlist[pl.ds(i * 16, 16)] = v >> 9
        molist[pl.ds(i * 16, 16)] = v & 511
        return 0
    lax.fori_loop(0, CAP // 16, unpack_body, 0)

    cbuf[pl.ds(0, 16)] = jnp.full((16,), cnt, jnp.int32)
    pltpu.sync_copy(mlist, lists_hbm.at[w])
    pltpu.sync_copy(molist, offs_hbm.at[w])
    pltpu.sync_copy(cbuf, counts_hbm.at[w])


@functools.partial(
    pl.kernel,
    out_type=jax.ShapeDtypeStruct((E, D), jnp.float32),
    mesh=_mesh,
    scratch_types=[
        pltpu.VMEM((GCH,), jnp.int32),
        pltpu.VMEM((GCH,), jnp.int32),
        pltpu.VMEM((GCH, D), jnp.float32),
        pltpu.VMEM((GCH, D), jnp.float32),
        pltpu.SemaphoreType.DMA,
        pltpu.SemaphoreType.DMA,
        pltpu.SemaphoreType.DMA,
    ],
)
def _gather_add(pd_hbm, ps_hbm, dst_hbm, src_hbm, pre_hbm,
                idxd, idxs, rowsd, rowss, semd, sems, semw):
    w = _wid()
    base = pl.multiple_of(w * EPW, 8)

    def chunk_body(k, _):
        off = pl.multiple_of(base + k * GCH, 8)
        pltpu.sync_copy(dst_hbm.at[pl.ds(off, GCH)], idxd)
        pltpu.sync_copy(src_hbm.at[pl.ds(off, GCH)], idxs)
        cpd = pltpu.async_copy(pd_hbm.at[idxd], rowsd, semd)
        cps = pltpu.async_copy(ps_hbm.at[idxs], rowss, sems)
        cpd.wait()
        cps.wait()

        @pl.when(k > 0)
        def _():
            pltpu.make_async_copy(
                rowsd, pre_hbm.at[pl.ds(off, GCH)], semw).wait()

        def add_body(i, _):
            for j in range(D // 16):
                rowsd[i, pl.ds(j * 16, 16)] = (
                    rowsd[i, pl.ds(j * 16, 16)] + rowss[i, pl.ds(j * 16, 16)])
            return 0
        lax.fori_loop(0, GCH, add_body, 0)
        pltpu.async_copy(rowsd, pre_hbm.at[pl.ds(off, GCH)], semw)
        return 0
    lax.fori_loop(0, NGCH, chunk_body, 0)
    pltpu.make_async_copy(
        rowsd, pre_hbm.at[pl.ds(base, GCH)], semw).wait()


@functools.partial(
    pl.kernel,
    out_type=jax.ShapeDtypeStruct((N, D), jnp.float32),
    mesh=_mesh,
    scratch_types=[
        pltpu.VMEM((NPW + 1, D), jnp.float32),
        pltpu.VMEM((SCH,), jnp.int32),
        pltpu.VMEM((SCH,), jnp.int32),
        pltpu.VMEM((SCH, D), jnp.float32),
        pltpu.VMEM((16,), jnp.int32),
        pltpu.SemaphoreType.DMA,
    ],
)
def _scatter_max(m2_hbm, lists_hbm, offs_hbm, counts_hbm, agg_hbm,
                 table, idxb, offb, rows, cbuf, sem):
    w = _wid()

    def zero_body(r, _):
        for j in range(D // 16):
            table[r, pl.ds(j * 16, 16)] = jnp.zeros((16,), jnp.float32)
        return 0
    lax.fori_loop(0, NPW + 1, zero_body, 0)

    pltpu.sync_copy(counts_hbm.at[w], cbuf)
    cnt = cbuf[pl.ds(0, 16)][0]
    nchunks = (cnt + SCH - 1) // SCH

    def chunk_body(k, _):
        cb = pl.multiple_of(k * SCH, 8)
        pltpu.sync_copy(lists_hbm.at[w, pl.ds(cb, SCH)], idxb)
        pltpu.sync_copy(offs_hbm.at[w, pl.ds(cb, SCH)], offb)
        pltpu.async_copy(m2_hbm.at[idxb], rows, sem).wait()

        def group_body(g, _):
            off_vec = offb[pl.ds(g * 16, 16)]
            for l in range(16):
                off = off_vec[l]
                i = g * 16 + l
                for j in range(D // 16):
                    cur = table[off, pl.ds(j * 16, 16)]
                    seg = rows[i, pl.ds(j * 16, 16)]
                    table[off, pl.ds(j * 16, 16)] = jnp.maximum(cur, seg)
            return 0
        lax.fori_loop(0, SCH // 16, group_body, 0)
        return 0
    lax.fori_loop(0, nchunks, chunk_body, 0)

    lo = pl.multiple_of(w * NPW, 8)

    @pl.when(w < NW - 1)
    def _():
        pltpu.sync_copy(table.at[pl.ds(0, NPW)], agg_hbm.at[pl.ds(lo, NPW)])

    @pl.when(w == NW - 1)
    def _():
        pltpu.sync_copy(table.at[pl.ds(0, N - (NW - 1) * NPW)],
                        agg_hbm.at[pl.ds(lo, N - (NW - 1) * NPW)])


def _node_transform_body(h_ref, wa1_ref, wa2_ref, pd_ref, ps_ref):
    h = h_ref[...]
    a = jnp.dot(h, wa1_ref[...], preferred_element_type=jnp.float32)
    b = jnp.dot(h, wa2_ref[...], preferred_element_type=jnp.float32)
    pd_ref[...] = a - b
    ps_ref[...] = b


def _node_transform(h, wa1, wa2):
    bn = 1000
    grid = N // bn
    return pl.pallas_call(
        _node_transform_body,
        grid=(grid,),
        in_specs=[
            pl.BlockSpec((bn, D), lambda i: (i, 0)),
            pl.BlockSpec((D, D), lambda i: (0, 0)),
            pl.BlockSpec((D, D), lambda i: (0, 0)),
        ],
        out_specs=[
            pl.BlockSpec((bn, D), lambda i: (i, 0)),
            pl.BlockSpec((bn, D), lambda i: (i, 0)),
        ],
        out_shape=[
            jax.ShapeDtypeStruct((N, D), jnp.float32),
            jax.ShapeDtypeStruct((N, D), jnp.float32),
        ],
    )(h, wa1, wa2)


def _edge_mlp_body(pre_ref, ea_ref, wa3_ref, ba_ref, wb_ref, bb_ref, out_ref):
    ea = jnp.dot(ea_ref[...], wa3_ref[...], preferred_element_type=jnp.float32)
    a = jax.nn.relu(pre_ref[...] + ea + ba_ref[...])
    out = jnp.dot(a, wb_ref[...], preferred_element_type=jnp.float32)
    out_ref[...] = jax.nn.relu(out + bb_ref[...])


def _edge_mlp(pre, edge_attr, wa3, ba, wb, bb):
    be = 640
    grid = E // be
    ba2 = ba.reshape(1, -1)
    bb2 = bb.reshape(1, -1)
    return pl.pallas_call(
        _edge_mlp_body,
        grid=(grid,),
        in_specs=[
            pl.BlockSpec((be, D), lambda i: (i, 0)),
            pl.BlockSpec((be, DE), lambda i: (i, 0)),
            pl.BlockSpec((DE, D), lambda i: (0, 0)),
            pl.BlockSpec((1, D), lambda i: (0, 0)),
            pl.BlockSpec((D, D), lambda i: (0, 0)),
            pl.BlockSpec((1, D), lambda i: (0, 0)),
        ],
        out_specs=pl.BlockSpec((be, D), lambda i: (i, 0)),
        out_shape=jax.ShapeDtypeStruct((E, D), jnp.float32),
    )(pre, edge_attr, wa3, ba2, wb, bb2)


def _head_body(h_ref, w3_ref, b3_ref, w4_ref, b4_ref, out_ref):
    a = jax.nn.relu(
        jnp.dot(h_ref[...], w3_ref[...], preferred_element_type=jnp.float32)
        + b3_ref[...])
    o = jnp.dot(a, w4_ref[...], preferred_element_type=jnp.float32) + b4_ref[...]
    out_ref[...] = jax.nn.sigmoid(o)


def _head(h, w3, b3, w4, b4):
    bn = 1000
    grid = N // bn
    h2 = w3.shape[1]
    out = w4.shape[1]
    return pl.pallas_call(
        _head_body,
        grid=(grid,),
        in_specs=[
            pl.BlockSpec((bn, D), lambda i: (i, 0)),
            pl.BlockSpec((D, h2), lambda i: (0, 0)),
            pl.BlockSpec((1, h2), lambda i: (0, 0)),
            pl.BlockSpec((h2, out), lambda i: (0, 0)),
            pl.BlockSpec((1, out), lambda i: (0, 0)),
        ],
        out_specs=pl.BlockSpec((bn, out), lambda i: (i, 0)),
        out_shape=jax.ShapeDtypeStruct((N, out), jnp.float32),
    )(h, w3, b3.reshape(1, -1), w4, b4.reshape(1, -1))


def _layer(h, src, dst, edge_attr, wa, ba, wb, bb, lists, offs, counts):
    hin = h.shape[1]
    wa1 = wa[:hin]
    wa2 = wa[hin:2 * hin]
    wa3 = wa[2 * hin:]
    pd, ps = _node_transform(h, wa1, wa2)
    pre = _gather_add(pd, ps, dst, src)
    m2 = _edge_mlp(pre, edge_attr, wa3, ba, wb, bb)
    return _scatter_max(m2, lists, offs, counts)


def kernel(x, edge_index, edge_attr, W1a, b1a, W1b, b1b, W2a, b2a, W2b, b2b,
           W3, b3, W4, b4):
    src = edge_index[0]
    dst = edge_index[1]
    lists, offs, counts = _bucketize(dst)
    h = _layer(x, src, dst, edge_attr, W1a, b1a, W1b, b1b, lists, offs, counts)
    h = _layer(h, src, dst, edge_attr, W2a, b2a, W2b, b2b, lists, offs, counts)
    return _head(h, W3, b3, W4, b4)
